# Initial kernel scaffold; baseline (speedup 1.0000x reference)
import functools

import jax
import jax.numpy as jnp
from jax import lax
from jax.experimental import pallas as pl
from jax.experimental.pallas import tpu as pltpu
from jax.experimental.pallas import tpu_sc as plsc

_D = 768
_MAX_REL = 32
_S = 512
_NROWS = 1024


def _rpe_sc_kernel(table_hbm, out_hbm, idx_v, rows_v, drev_sh, gsem, osem):
    nc = 2
    cid = lax.axis_index("c")
    sid = lax.axis_index("s")

    base = sid * (_NROWS // 16)
    lane = lax.iota(jnp.int32, 16)
    for j in range(4):
        p = base + j * 16 + lane
        idx = jnp.clip(511 - p, -_MAX_REL, _MAX_REL) + _MAX_REL
        idx_v[pl.ds(j * 16, 16)] = idx
    pltpu.async_copy(table_hbm.at[idx_v], rows_v, gsem).wait()
    pltpu.sync_copy(rows_v, drev_sh.at[pl.ds(base, _NROWS // 16)])
    plsc.subcore_barrier()

    wid = sid * nc + cid
    copies = []
    for t in range(16):
        i = wid * 16 + t
        start = 511 - i
        copies.append(
            pltpu.async_copy(drev_sh.at[pl.ds(start, _S)], out_hbm.at[i], osem)
        )
    for c in copies:
        c.wait()


def kernel(table, seq_len):
    del seq_len
    mesh = plsc.VectorSubcoreMesh(core_axis_name="c", subcore_axis_name="s")
    k = functools.partial(
        pl.kernel,
        mesh=mesh,
        out_type=jax.ShapeDtypeStruct((_S, _S, _D), jnp.float32),
        scratch_types=[
            pltpu.VMEM((64,), jnp.int32),
            pltpu.VMEM((64, _D), jnp.float32),
            pltpu.VMEM_SHARED((_NROWS, _D), jnp.float32),
            pltpu.SemaphoreType.DMA,
            pltpu.SemaphoreType.DMA,
        ],
    )(_rpe_sc_kernel)
    return k(table)


# speedup vs baseline: 1.5645x; 1.5645x over previous
import functools

import jax
import jax.numpy as jnp
from jax import lax
from jax.experimental import pallas as pl
from jax.experimental.pallas import tpu as pltpu
from jax.experimental.pallas import tpu_sc as plsc

_D = 768
_MAX_REL = 32
_S = 512
_NROWS = 1024


def _rpe_sc_kernel(table_hbm, out_hbm, drev_sh, gsem, osem):
    nc = 2
    cid = lax.axis_index("c")
    sid = lax.axis_index("s")

    base = sid * (_NROWS // 16)
    fills = []
    for m in range(_NROWS // 16):
        p = base + m
        ridx = jnp.clip(511 - p, -_MAX_REL, _MAX_REL) + _MAX_REL
        fills.append(
            pltpu.async_copy(
                table_hbm.at[ridx], drev_sh.at[pl.ds(p * _D, _D)], gsem
            )
        )
    for f in fills:
        f.wait()
    plsc.subcore_barrier()

    wid = sid * nc + cid
    copies = []
    for t in range(16):
        i = wid * 16 + t
        start = (511 - i) * _D
        copies.append(
            pltpu.async_copy(
                drev_sh.at[pl.ds(start, _S * _D)], out_hbm.at[i], osem
            )
        )
    for c in copies:
        c.wait()


def kernel(table, seq_len):
    del seq_len
    mesh = plsc.VectorSubcoreMesh(core_axis_name="c", subcore_axis_name="s")
    k = functools.partial(
        pl.kernel,
        mesh=mesh,
        out_type=jax.ShapeDtypeStruct((_S, _S * _D), jnp.float32),
        scratch_types=[
            pltpu.VMEM_SHARED((_NROWS * _D,), jnp.float32),
            pltpu.SemaphoreType.DMA,
            pltpu.SemaphoreType.DMA,
        ],
    )(_rpe_sc_kernel)
    flat = k(table)
    return flat.reshape(_S, _S, _D)


# speedup vs baseline: 3.1462x; 2.0110x over previous
import functools

import jax
import jax.numpy as jnp
from jax import lax
from jax.experimental import pallas as pl
from jax.experimental.pallas import tpu as pltpu
from jax.experimental.pallas import tpu_sc as plsc

_D = 768
_MAX_REL = 32
_S = 512
_CROWS = 128
_PIECES = (8, 4, 2, 1)


def _clip_idx(p):
    return jnp.clip(511 - p, -_MAX_REL, _MAX_REL) + _MAX_REL


def _rpe_sc_kernel(table_hbm, out_hbm, idx_v, stage_v, c64_sh, c0_sh, b3_sh,
                   gsem, osem):
    nc = 2
    cid = lax.axis_index("c")
    sid = lax.axis_index("s")
    lane = lax.iota(jnp.int32, 16)

    for u in range(8):
        b0 = sid * 16 + 2 * u
        r = lax.shift_right_logical(sid, 1)
        m0 = (sid & 1) * 16 + 2 * u
        p = 384 + r + 8 * m0 + lane
        idx_v[pl.ds(0, 16)] = _clip_idx(p)
        pltpu.async_copy(table_hbm.at[idx_v], stage_v, gsem).wait()
        pltpu.sync_copy(stage_v.at[pl.ds(0, 8)], b3_sh.at[b0])
        pltpu.sync_copy(stage_v.at[pl.ds(8, 8)], b3_sh.at[b0 + 1])

    @pl.when(sid < 2)
    def _():
        idx_v[pl.ds(0, 16)] = jnp.where(sid == 0, 64, 0) + 0 * lane
        pltpu.async_copy(table_hbm.at[idx_v], stage_v, gsem).wait()

    @pl.when(sid == 0)
    def _():
        for g in range(_CROWS // 16):
            pltpu.sync_copy(stage_v, c64_sh.at[pl.ds(16 * g, 16)])

    @pl.when(sid == 1)
    def _():
        for g in range(_CROWS // 16):
            pltpu.sync_copy(stage_v, c0_sh.at[pl.ds(16 * g, 16)])

    plsc.subcore_barrier()

    wid = sid * nc + cid

    def do_slab(t, carry):
        i = wid * 16 + t
        s = 511 - i
        r = s & 7
        g_head = jnp.clip(lax.shift_right_arithmetic(480 - s, 3), 0, 48)
        g_tail = 48 - g_head
        b0 = 32 * r + lax.shift_right_arithmetic(s, 3) + g_head - 48

        def pieces(n_groups, dst_group0, src_sh):
            out = []
            for q in range(3):
                out.append((
                    n_groups >= 16 * (q + 1),
                    src_sh.at[pl.ds(0, 128)],
                    out_hbm.at[i, pl.ds(8 * (dst_group0 + 16 * q), 128)],
                ))
            rem_off = dst_group0 + (n_groups & 48)
            for p_sz in _PIECES:
                off = rem_off + (n_groups & (16 - 2 * p_sz))
                out.append((
                    (n_groups & p_sz) != 0,
                    src_sh.at[pl.ds(0, 8 * p_sz)],
                    out_hbm.at[i, pl.ds(8 * off, 8 * p_sz)],
                ))
            return out

        plan = pieces(g_head, 0, c64_sh) + pieces(g_tail, g_head + 16, c0_sh)
        for cond, src, dst in plan:
            @pl.when(cond)
            def _():
                pltpu.async_copy(src, dst, osem)
        band = []
        for k in range(16):
            band.append((
                b3_sh.at[b0 + k],
                out_hbm.at[i, pl.ds(8 * (g_head + k), 8)],
            ))
        for src, dst in band:
            pltpu.async_copy(src, dst, osem)
        for cond, src, dst in plan:
            @pl.when(cond)
            def _():
                pltpu.make_async_copy(src, dst, osem).wait()
        for src, dst in band:
            pltpu.make_async_copy(src, dst, osem).wait()
        return carry

    lax.fori_loop(0, 16, do_slab, 0)


def kernel(table, seq_len):
    del seq_len
    mesh = plsc.VectorSubcoreMesh(core_axis_name="c", subcore_axis_name="s")
    k = functools.partial(
        pl.kernel,
        mesh=mesh,
        out_type=jax.ShapeDtypeStruct((_S, _S, _D), jnp.float32),
        scratch_types=[
            pltpu.VMEM((16,), jnp.int32),
            pltpu.VMEM((16, _D), jnp.float32),
            pltpu.VMEM_SHARED((_CROWS, _D), jnp.float32),
            pltpu.VMEM_SHARED((_CROWS, _D), jnp.float32),
            pltpu.VMEM_SHARED((256, 8, _D), jnp.float32),
            pltpu.SemaphoreType.DMA,
            pltpu.SemaphoreType.DMA,
        ],
    )(_rpe_sc_kernel)
    return k(table)


# speedup vs baseline: 3.1732x; 1.0086x over previous
import functools

import jax
import jax.numpy as jnp
from jax import lax
from jax.experimental import pallas as pl
from jax.experimental.pallas import tpu as pltpu
from jax.experimental.pallas import tpu_sc as plsc

_D = 768
_MAX_REL = 32
_S = 512
_CROWS = 128
_PIECES = (8, 4, 2, 1)


def _clip_idx(p):
    return jnp.clip(511 - p, -_MAX_REL, _MAX_REL) + _MAX_REL


def _rpe_sc_kernel(table_hbm, out_hbm, idx_v, stage_v, c64_sh, c0_sh, b3_sh,
                   gsem, osem):
    nc = 2
    cid = lax.axis_index("c")
    sid = lax.axis_index("s")
    lane = lax.iota(jnp.int32, 16)

    for u in range(8):
        b0 = sid * 16 + 2 * u
        r = lax.shift_right_logical(sid, 1)
        m0 = (sid & 1) * 16 + 2 * u
        p = 384 + r + 8 * m0 + lane
        idx_v[pl.ds(0, 16)] = _clip_idx(p)
        pltpu.async_copy(table_hbm.at[idx_v], stage_v, gsem).wait()
        pltpu.sync_copy(stage_v, b3_sh.at[pl.ds(8 * b0, 16)])

    @pl.when(sid < 2)
    def _():
        idx_v[pl.ds(0, 16)] = jnp.where(sid == 0, 64, 0) + 0 * lane
        pltpu.async_copy(table_hbm.at[idx_v], stage_v, gsem).wait()

    @pl.when(sid == 0)
    def _():
        for g in range(_CROWS // 16):
            pltpu.sync_copy(stage_v, c64_sh.at[pl.ds(16 * g, 16)])

    @pl.when(sid == 1)
    def _():
        for g in range(_CROWS // 16):
            pltpu.sync_copy(stage_v, c0_sh.at[pl.ds(16 * g, 16)])

    plsc.subcore_barrier()

    wid = sid * nc + cid

    def make_plan(t):
        i = wid * 16 + t
        s = 511 - i
        r = s & 7
        g_head = jnp.clip(lax.shift_right_arithmetic(480 - s, 3), 0, 48)
        g_tail = 48 - g_head
        b0 = 32 * r + lax.shift_right_arithmetic(s, 3) + g_head - 48

        def pieces(n_groups, dst_group0, src_sh):
            out = []
            for q in range(3):
                out.append((
                    n_groups >= 16 * (q + 1),
                    src_sh.at[pl.ds(0, 128)],
                    out_hbm.at[i, pl.ds(8 * (dst_group0 + 16 * q), 128)],
                ))
            rem_off = dst_group0 + (n_groups & 48)
            for p_sz in _PIECES:
                off = rem_off + (n_groups & (16 - 2 * p_sz))
                out.append((
                    (n_groups & p_sz) != 0,
                    src_sh.at[pl.ds(0, 8 * p_sz)],
                    out_hbm.at[i, pl.ds(8 * off, 8 * p_sz)],
                ))
            return out

        plan = pieces(g_head, 0, c64_sh)
        plan.append((
            None,
            b3_sh.at[pl.ds(8 * b0, 128)],
            out_hbm.at[i, pl.ds(8 * g_head, 128)],
        ))
        plan += pieces(g_tail, g_head + 16, c0_sh)
        return plan

    def fire(t):
        for cond, src, dst in make_plan(t):
            if cond is None:
                pltpu.async_copy(src, dst, osem)
            else:
                @pl.when(cond)
                def _():
                    pltpu.async_copy(src, dst, osem)

    def drain(t):
        for cond, src, dst in make_plan(t):
            if cond is None:
                pltpu.make_async_copy(src, dst, osem).wait()
            else:
                @pl.when(cond)
                def _():
                    pltpu.make_async_copy(src, dst, osem).wait()

    def do_slab(t, carry):
        fire(t)

        @pl.when(t > 0)
        def _():
            drain(t - 1)

        return carry

    lax.fori_loop(0, 16, do_slab, 0)
    drain(15)


def kernel(table, seq_len):
    del seq_len
    mesh = plsc.VectorSubcoreMesh(core_axis_name="c", subcore_axis_name="s")
    k = functools.partial(
        pl.kernel,
        mesh=mesh,
        out_type=jax.ShapeDtypeStruct((_S, _S, _D), jnp.float32),
        scratch_types=[
            pltpu.VMEM((16,), jnp.int32),
            pltpu.VMEM((16, _D), jnp.float32),
            pltpu.VMEM_SHARED((_CROWS, _D), jnp.float32),
            pltpu.VMEM_SHARED((_CROWS, _D), jnp.float32),
            pltpu.VMEM_SHARED((2048, _D), jnp.float32),
            pltpu.SemaphoreType.DMA,
            pltpu.SemaphoreType.DMA,
        ],
    )(_rpe_sc_kernel)
    return k(table)
